# Initial kernel scaffold; baseline (speedup 1.0000x reference)
import jax
import jax.numpy as jnp
from jax.experimental import pallas as pl


def kernel(x, emb, pe):
    raise NotImplementedError("write your pallas kernel here")


# speedup vs baseline: 1.5907x; 1.5907x over previous
import functools

import jax
import jax.numpy as jnp
from jax import lax
from jax.experimental import pallas as pl
from jax.experimental.pallas import tpu as pltpu
from jax.experimental.pallas import tpu_sc as plsc

MAX_SEQ = 8192
D_MODEL = 768
BATCH = 4

NUM_CORES = 2
NUM_SUBCORES = 16
NUM_WORKERS = NUM_CORES * NUM_SUBCORES
S_PER_W = MAX_SEQ // NUM_WORKERS
CHUNK = 128
N_CHUNKS = S_PER_W // CHUNK

_MESH = plsc.VectorSubcoreMesh(core_axis_name="c", subcore_axis_name="s")


@functools.partial(
    pl.kernel,
    mesh=_MESH,
    out_type=jax.ShapeDtypeStruct((BATCH * MAX_SEQ, D_MODEL), jnp.float32),
    scratch_types=[
        pltpu.VMEM((CHUNK,), jnp.int32),
        pltpu.VMEM((CHUNK, D_MODEL), jnp.float32),
        pltpu.SemaphoreType.DMA,
    ],
)
def _pe_lookup_tile(emb_hbm, pe_hbm, out_hbm, idx_v, rows_v, sem):
    wid = lax.axis_index("s") * NUM_CORES + lax.axis_index("c")
    base = wid * S_PER_W
    for i in range(N_CHUNKS):
        off = base + i * CHUNK
        pltpu.sync_copy(pe_hbm.at[pl.ds(off, CHUNK)], idx_v)
        pltpu.async_copy(emb_hbm.at[idx_v], rows_v, sem).wait()
        for b in range(BATCH):
            pltpu.sync_copy(rows_v, out_hbm.at[pl.ds(b * MAX_SEQ + off, CHUNK)])


def kernel(x, emb, pe):
    del x
    pe_flat = pe.reshape(MAX_SEQ).astype(jnp.int32)
    out = _pe_lookup_tile(emb, pe_flat)
    return out.reshape(BATCH, MAX_SEQ, D_MODEL)
